# Initial kernel scaffold; baseline (speedup 1.0000x reference)
import jax
import jax.numpy as jnp
from jax.experimental import pallas as pl


def kernel(S, J, C, D, P):
    raise NotImplementedError("write your pallas kernel here")


# speedup vs baseline: 1.5440x; 1.5440x over previous
import functools

import jax
import jax.numpy as jnp
import numpy as np
from jax import lax
from jax.experimental import pallas as pl
from jax.experimental.pallas import tpu as pltpu
from jax.experimental.pallas import tpu_sc as plsc

N = 20000
G = 5000
NC = 81
ND = 324

NW = 32
SEGB = 160
RCH = 64
NCB = NC // 16
NDB = ND // 16


CTAIL = NC - 16
DTAIL = ND - 16


def _sc_body(S_hbm, J_hbm, C_hbm, D_hbm, bounds_hbm,
             MC_hbm, MD_hbm,
             bounds_v, Sv, Jv, Cv, Dv, AC, AD, ws_sm):
    c = lax.axis_index("c")
    s = lax.axis_index("s")
    wid = s * 2 + c
    g0 = pl.multiple_of(jnp.minimum(wid * SEGB, G - SEGB), 8)
    zero16 = jnp.zeros((16,), jnp.float32)
    lane = lax.iota(jnp.int32, 16)
    cmask = lane == 0
    ccol = jnp.full((16,), NC - 1, jnp.int32)
    dmask = lane < (ND - 16 * NDB)
    dcol = jnp.minimum(16 * NDB + lane, ND - 1)

    def zrow(g, carry):
        gv = jnp.full((16,), g, jnp.int32)
        for k in range(NCB):
            AC[g, pl.ds(k * 16, 16)] = zero16
        plsc.store_scatter(AC, [gv, ccol], zero16, mask=cmask)
        for k in range(NDB):
            AD[g, pl.ds(k * 16, 16)] = zero16
        plsc.store_scatter(AD, [gv, dcol], zero16, mask=dmask)
        ws_sm[g] = 0.0
        return carry
    lax.fori_loop(0, SEGB, zrow, 0)

    pltpu.sync_copy(bounds_hbm, bounds_v)
    b16 = bounds_v[wid, :]
    lo = b16[0]
    hi = b16[1]
    lo8 = lo - lax.rem(lo, 8)
    nch = (hi - lo8 + (RCH - 1)) // RCH

    def chunk(cix, carry):
        r0 = lo8 + cix * RCH
        b = pl.multiple_of(jnp.minimum(r0, N - RCH), 8)
        o = r0 - b
        pltpu.sync_copy(S_hbm.at[pl.ds(b, RCH)], Sv)
        pltpu.sync_copy(J_hbm.at[pl.ds(b, RCH)], Jv)
        pltpu.sync_copy(C_hbm.at[pl.ds(b, RCH), :], Cv)
        pltpu.sync_copy(D_hbm.at[pl.ds(b, RCH), :], Dv)

        def grp(t, gcarry):
            base = pl.multiple_of(t * 16, 8)
            jv16 = Jv[pl.ds(base, 16)]
            sv16 = Sv[pl.ds(base, 16)]
            rowid = base + lane
            inr = (jv16 >= g0) & (jv16 < g0 + SEGB) & (rowid >= o)
            sv16m = jnp.where(inr, sv16, 0.0)
            sidx16 = jnp.clip(jv16 - g0, 0, SEGB - 1)
            for l in range(16):
                sidx = sidx16[l]
                sidxv = jnp.full((16,), sidx)
                svec = jnp.full((16,), sv16m[l])
                i = base + l
                iv = jnp.full((16,), i, jnp.int32)
                ws_sm[sidx] = ws_sm[sidx] + sv16m[l]
                for k in range(NCB):
                    ksl = pl.ds(k * 16, 16)
                    AC[sidx, ksl] = AC[sidx, ksl] + Cv[i, ksl] * svec
                xc = plsc.load_gather(Cv, [iv, ccol], mask=cmask)
                plsc.addupdate_scatter(AC, [sidxv, ccol], xc * svec, mask=cmask)
                for k in range(NDB):
                    ksl = pl.ds(k * 16, 16)
                    AD[sidx, ksl] = AD[sidx, ksl] + Dv[i, ksl] * svec
                xd = plsc.load_gather(Dv, [iv, dcol], mask=dmask)
                plsc.addupdate_scatter(AD, [sidxv, dcol], xd * svec, mask=dmask)
            return gcarry
        lax.fori_loop(0, RCH // 16, grp, 0)
        return carry
    lax.fori_loop(0, nch, chunk, 0)

    def drow(g, carry):
        gv = jnp.full((16,), g, jnp.int32)
        rv = 1.0 / (jnp.full((16,), ws_sm[g]) + 1e-6)
        for k in range(NCB):
            ksl = pl.ds(k * 16, 16)
            AC[g, ksl] = AC[g, ksl] * rv
        xc = plsc.load_gather(AC, [gv, ccol], mask=cmask)
        plsc.store_scatter(AC, [gv, ccol], xc * rv, mask=cmask)
        for k in range(NDB):
            ksl = pl.ds(k * 16, 16)
            AD[g, ksl] = AD[g, ksl] * rv
        xd = plsc.load_gather(AD, [gv, dcol], mask=dmask)
        plsc.store_scatter(AD, [gv, dcol], xd * rv, mask=dmask)
        return carry
    lax.fori_loop(0, SEGB, drow, 0)

    pltpu.sync_copy(AC, MC_hbm.at[pl.ds(g0, SEGB), :])
    pltpu.sync_copy(AD, MD_hbm.at[pl.ds(g0, SEGB), :])


@jax.jit
def _roimerge_sc(S, J32, C, D, bounds):
    mesh = plsc.VectorSubcoreMesh(core_axis_name="c", subcore_axis_name="s")
    run = functools.partial(
        pl.kernel,
        out_type=(
            jax.ShapeDtypeStruct((G, NC), jnp.float32),
            jax.ShapeDtypeStruct((G, ND), jnp.float32),
        ),
        mesh=mesh,
        scratch_types=[
            pltpu.VMEM((NW, 16), jnp.int32),
            pltpu.VMEM((RCH,), jnp.float32),
            pltpu.VMEM((RCH,), jnp.int32),
            pltpu.VMEM((RCH, NC), jnp.float32),
            pltpu.VMEM((RCH, ND), jnp.float32),
            pltpu.VMEM((SEGB, NC), jnp.float32),
            pltpu.VMEM((SEGB, ND), jnp.float32),
            pltpu.SMEM((SEGB,), jnp.float32),
        ],
        compiler_params=pltpu.CompilerParams(needs_layout_passes=False),
    )(_sc_body)
    return run(S, J32, C, D, bounds)


def kernel(S, J, C, D, P):
    J32 = J.astype(jnp.int32)
    g0s = np.minimum(np.arange(NW, dtype=np.int32) * SEGB, G - SEGB).astype(np.int32)
    qs = jnp.asarray(np.stack([g0s, g0s + SEGB], axis=1).astype(np.int32))
    lohi = jnp.searchsorted(J32, qs.reshape(-1), side="left").astype(jnp.int32)
    bounds = jnp.zeros((NW, 16), jnp.int32).at[:, :2].set(lohi.reshape(NW, 2))
    MC, MD = _roimerge_sc(S, J32, C, D, bounds)
    return (MC, MD)


# speedup vs baseline: 1.7716x; 1.1474x over previous
import functools

import jax
import jax.numpy as jnp
import numpy as np
from jax import lax
from jax.experimental import pallas as pl
from jax.experimental.pallas import tpu as pltpu
from jax.experimental.pallas import tpu_sc as plsc

N = 20000
G = 5000
NC = 81
ND = 324

NW = 32
SEGB = 160
RCH = 64
NCB = NC // 16
NDB = ND // 16


CTAIL = NC - 16
DTAIL = ND - 16


def _sc_body(S_hbm, J_hbm, C_hbm, D_hbm, bounds_hbm,
             MC_hbm, MD_hbm,
             bounds_v, Sv, Jv, Cv, Dv, AC, AD, ws_v):
    c = lax.axis_index("c")
    s = lax.axis_index("s")
    wid = s * 2 + c
    g0 = pl.multiple_of(jnp.minimum(wid * SEGB, G - SEGB), 8)
    zero16 = jnp.zeros((16,), jnp.float32)
    lane = lax.iota(jnp.int32, 16)
    cmask = lane == 0
    ccol = jnp.full((16,), NC - 1, jnp.int32)
    dmask = lane < (ND - 16 * NDB)
    dcol = jnp.minimum(16 * NDB + lane, ND - 1)

    def zrow(g, carry):
        gv = jnp.full((16,), g, jnp.int32)
        for k in range(NCB):
            AC[g, pl.ds(k * 16, 16)] = zero16
        plsc.store_scatter(AC, [gv, ccol], zero16, mask=cmask)
        for k in range(NDB):
            AD[g, pl.ds(k * 16, 16)] = zero16
        plsc.store_scatter(AD, [gv, dcol], zero16, mask=dmask)
        return carry
    lax.fori_loop(0, SEGB, zrow, 0)
    for t in range(SEGB // 16):
        ws_v[pl.ds(t * 16, 16)] = zero16

    pltpu.sync_copy(bounds_hbm, bounds_v)
    b16 = bounds_v[wid, :]
    lo = b16[0]
    hi = b16[1]
    lo8 = lo - lax.rem(lo, 8)
    nch = (hi - lo8 + (RCH - 1)) // RCH

    def chunk(cix, carry):
        r0 = lo8 + cix * RCH
        b = pl.multiple_of(jnp.minimum(r0, N - RCH), 8)
        o = r0 - b
        pltpu.sync_copy(S_hbm.at[pl.ds(b, RCH)], Sv)
        pltpu.sync_copy(J_hbm.at[pl.ds(b, RCH)], Jv)
        pltpu.sync_copy(C_hbm.at[pl.ds(b, RCH), :], Cv)
        pltpu.sync_copy(D_hbm.at[pl.ds(b, RCH), :], Dv)

        def grp(t, gcarry):
            base = pl.multiple_of(t * 16, 8)
            jv16 = Jv[pl.ds(base, 16)]
            sv16 = Sv[pl.ds(base, 16)]
            rowid = base + lane
            inr = (jv16 >= g0) & (jv16 < g0 + SEGB) & (rowid >= o)
            sv16m = jnp.where(inr, sv16, 0.0)
            sidx16 = jnp.clip(jv16 - g0, 0, SEGB - 1)
            plsc.addupdate_scatter(ws_v, [sidx16], sv16m)
            for l in range(16):
                sidxv = jnp.full((16,), sidx16[l])
                svec = jnp.full((16,), sv16m[l])
                i = base + l
                iv = jnp.full((16,), i, jnp.int32)
                for k in range(NCB):
                    plsc.addupdate_scatter(AC, [sidxv, lane + (k * 16)],
                                           Cv[i, pl.ds(k * 16, 16)] * svec)
                xc = plsc.load_gather(Cv, [iv, ccol], mask=cmask)
                plsc.addupdate_scatter(AC, [sidxv, ccol], xc * svec, mask=cmask)
                for k in range(NDB):
                    plsc.addupdate_scatter(AD, [sidxv, lane + (k * 16)],
                                           Dv[i, pl.ds(k * 16, 16)] * svec)
                xd = plsc.load_gather(Dv, [iv, dcol], mask=dmask)
                plsc.addupdate_scatter(AD, [sidxv, dcol], xd * svec, mask=dmask)
            return gcarry
        lax.fori_loop(0, RCH // 16, grp, 0)
        return carry
    lax.fori_loop(0, nch, chunk, 0)

    def drow(g, carry):
        gv = jnp.full((16,), g, jnp.int32)
        rv = 1.0 / (plsc.load_gather(ws_v, [gv]) + 1e-6)
        for k in range(NCB):
            ksl = pl.ds(k * 16, 16)
            AC[g, ksl] = AC[g, ksl] * rv
        xc = plsc.load_gather(AC, [gv, ccol], mask=cmask)
        plsc.store_scatter(AC, [gv, ccol], xc * rv, mask=cmask)
        for k in range(NDB):
            ksl = pl.ds(k * 16, 16)
            AD[g, ksl] = AD[g, ksl] * rv
        xd = plsc.load_gather(AD, [gv, dcol], mask=dmask)
        plsc.store_scatter(AD, [gv, dcol], xd * rv, mask=dmask)
        return carry
    lax.fori_loop(0, SEGB, drow, 0)

    pltpu.sync_copy(AC, MC_hbm.at[pl.ds(g0, SEGB), :])
    pltpu.sync_copy(AD, MD_hbm.at[pl.ds(g0, SEGB), :])


@jax.jit
def _roimerge_sc(S, J32, C, D, bounds):
    mesh = plsc.VectorSubcoreMesh(core_axis_name="c", subcore_axis_name="s")
    run = functools.partial(
        pl.kernel,
        out_type=(
            jax.ShapeDtypeStruct((G, NC), jnp.float32),
            jax.ShapeDtypeStruct((G, ND), jnp.float32),
        ),
        mesh=mesh,
        scratch_types=[
            pltpu.VMEM((NW, 16), jnp.int32),
            pltpu.VMEM((RCH,), jnp.float32),
            pltpu.VMEM((RCH,), jnp.int32),
            pltpu.VMEM((RCH, NC), jnp.float32),
            pltpu.VMEM((RCH, ND), jnp.float32),
            pltpu.VMEM((SEGB, NC), jnp.float32),
            pltpu.VMEM((SEGB, ND), jnp.float32),
            pltpu.VMEM((SEGB,), jnp.float32),
        ],
        compiler_params=pltpu.CompilerParams(needs_layout_passes=False),
    )(_sc_body)
    return run(S, J32, C, D, bounds)


def kernel(S, J, C, D, P):
    J32 = J.astype(jnp.int32)
    g0s = np.minimum(np.arange(NW, dtype=np.int32) * SEGB, G - SEGB).astype(np.int32)
    qs = jnp.asarray(np.stack([g0s, g0s + SEGB], axis=1).astype(np.int32))
    lohi = jnp.searchsorted(J32, qs.reshape(-1), side="left").astype(jnp.int32)
    bounds = jnp.zeros((NW, 16), jnp.int32).at[:, :2].set(lohi.reshape(NW, 2))
    MC, MD = _roimerge_sc(S, J32, C, D, bounds)
    return (MC, MD)


# speedup vs baseline: 2.9282x; 1.6528x over previous
import functools

import jax
import jax.numpy as jnp
import numpy as np
from jax import lax
from jax.experimental import pallas as pl
from jax.experimental.pallas import tpu as pltpu
from jax.experimental.pallas import tpu_sc as plsc

N = 20000
G = 5000
NC = 81
ND = 324

NW = 32
SEGB = 160
RCH = 64
NCB = NC // 16
NDB = ND // 16


CTAIL = NC - 16
DTAIL = ND - 16


def _sc_body(S_hbm, J_hbm, C_hbm, D_hbm, bounds_hbm,
             MC_hbm, MD_hbm,
             bounds_v, Sv, Jv, Cv, Dv, AC, AD, ws_v):
    c = lax.axis_index("c")
    s = lax.axis_index("s")
    wid = s * 2 + c
    g0 = pl.multiple_of(jnp.minimum(wid * SEGB, G - SEGB), 8)
    zero16 = jnp.zeros((16,), jnp.float32)
    lane = lax.iota(jnp.int32, 16)
    cmask = lane == 0
    ccol = jnp.full((16,), NC - 1, jnp.int32)
    dmask = lane < (ND - 16 * NDB)
    dcol = jnp.minimum(16 * NDB + lane, ND - 1)

    def zrow(g, carry):
        gv = jnp.full((16,), g, jnp.int32)
        for k in range(NCB):
            AC[g, pl.ds(k * 16, 16)] = zero16
        plsc.store_scatter(AC, [gv, ccol], zero16, mask=cmask)
        for k in range(NDB):
            AD[g, pl.ds(k * 16, 16)] = zero16
        plsc.store_scatter(AD, [gv, dcol], zero16, mask=dmask)
        return carry
    lax.fori_loop(0, SEGB, zrow, 0)
    for t in range(SEGB // 16):
        ws_v[pl.ds(t * 16, 16)] = zero16

    pltpu.sync_copy(bounds_hbm, bounds_v)
    b16 = bounds_v[wid, :]
    lo = b16[0]
    hi = b16[1]
    lo8 = lo - lax.rem(lo, 8)
    nch = (hi - lo8 + (RCH - 1)) // RCH

    def chunk(cix, carry):
        r0 = lo8 + cix * RCH
        b = pl.multiple_of(jnp.minimum(r0, N - RCH), 8)
        o = r0 - b
        pltpu.sync_copy(S_hbm.at[pl.ds(b, RCH)], Sv)
        pltpu.sync_copy(J_hbm.at[pl.ds(b, RCH)], Jv)
        pltpu.sync_copy(C_hbm.at[pl.ds(b, RCH), :], Cv)
        pltpu.sync_copy(D_hbm.at[pl.ds(b, RCH), :], Dv)

        def grp(t, gcarry):
            base = pl.multiple_of(t * 16, 8)
            jv16 = Jv[pl.ds(base, 16)]
            sv16 = Sv[pl.ds(base, 16)]
            rowid = base + lane
            inr = (jv16 >= g0) & (jv16 < g0 + SEGB) & (rowid >= o)
            sv16m = jnp.where(inr, sv16, 0.0)
            sidx16 = jnp.clip(jv16 - g0, 0, SEGB - 1)
            plsc.addupdate_scatter(ws_v, [sidx16], sv16m)
            for l in range(16):
                lv = jnp.full((16,), l, jnp.int32)
                sidxv = sidx16[lv]
                svec = sv16m[lv]
                i = base + l
                iv = jnp.full((16,), i, jnp.int32)
                xs_c = [Cv[i, pl.ds(k * 16, 16)] for k in range(NCB)]
                xs_d = [Dv[i, pl.ds(k * 16, 16)] for k in range(NDB)]
                xc = plsc.load_gather(Cv, [iv, ccol], mask=cmask)
                xd = plsc.load_gather(Dv, [iv, dcol], mask=dmask)
                ws_c = [x * svec for x in xs_c]
                ws_d = [x * svec for x in xs_d]
                for k in range(NCB):
                    plsc.addupdate_scatter(AC, [sidxv, lane + (k * 16)], ws_c[k])
                plsc.addupdate_scatter(AC, [sidxv, ccol], xc * svec, mask=cmask)
                for k in range(NDB):
                    plsc.addupdate_scatter(AD, [sidxv, lane + (k * 16)], ws_d[k])
                plsc.addupdate_scatter(AD, [sidxv, dcol], xd * svec, mask=dmask)
            return gcarry
        lax.fori_loop(0, RCH // 16, grp, 0)
        return carry
    lax.fori_loop(0, nch, chunk, 0)

    def drow(g, carry):
        gv = jnp.full((16,), g, jnp.int32)
        rv = 1.0 / (plsc.load_gather(ws_v, [gv]) + 1e-6)
        for k in range(NCB):
            ksl = pl.ds(k * 16, 16)
            AC[g, ksl] = AC[g, ksl] * rv
        xc = plsc.load_gather(AC, [gv, ccol], mask=cmask)
        plsc.store_scatter(AC, [gv, ccol], xc * rv, mask=cmask)
        for k in range(NDB):
            ksl = pl.ds(k * 16, 16)
            AD[g, ksl] = AD[g, ksl] * rv
        xd = plsc.load_gather(AD, [gv, dcol], mask=dmask)
        plsc.store_scatter(AD, [gv, dcol], xd * rv, mask=dmask)
        return carry
    lax.fori_loop(0, SEGB, drow, 0)

    pltpu.sync_copy(AC, MC_hbm.at[pl.ds(g0, SEGB), :])
    pltpu.sync_copy(AD, MD_hbm.at[pl.ds(g0, SEGB), :])


@jax.jit
def _roimerge_sc(S, J32, C, D, bounds):
    mesh = plsc.VectorSubcoreMesh(core_axis_name="c", subcore_axis_name="s")
    run = functools.partial(
        pl.kernel,
        out_type=(
            jax.ShapeDtypeStruct((G, NC), jnp.float32),
            jax.ShapeDtypeStruct((G, ND), jnp.float32),
        ),
        mesh=mesh,
        scratch_types=[
            pltpu.VMEM((NW, 16), jnp.int32),
            pltpu.VMEM((RCH,), jnp.float32),
            pltpu.VMEM((RCH,), jnp.int32),
            pltpu.VMEM((RCH, NC), jnp.float32),
            pltpu.VMEM((RCH, ND), jnp.float32),
            pltpu.VMEM((SEGB, NC), jnp.float32),
            pltpu.VMEM((SEGB, ND), jnp.float32),
            pltpu.VMEM((SEGB,), jnp.float32),
        ],
        compiler_params=pltpu.CompilerParams(needs_layout_passes=False),
    )(_sc_body)
    return run(S, J32, C, D, bounds)


def kernel(S, J, C, D, P):
    J32 = J.astype(jnp.int32)
    g0s = np.minimum(np.arange(NW, dtype=np.int32) * SEGB, G - SEGB).astype(np.int32)
    qs = jnp.asarray(np.stack([g0s, g0s + SEGB], axis=1).astype(np.int32))
    lohi = jnp.searchsorted(J32, qs.reshape(-1), side="left").astype(jnp.int32)
    bounds = jnp.zeros((NW, 16), jnp.int32).at[:, :2].set(lohi.reshape(NW, 2))
    MC, MD = _roimerge_sc(S, J32, C, D, bounds)
    return (MC, MD)


# speedup vs baseline: 3.5024x; 1.1961x over previous
import functools

import jax
import jax.numpy as jnp
import numpy as np
from jax import lax
from jax.experimental import pallas as pl
from jax.experimental.pallas import tpu as pltpu
from jax.experimental.pallas import tpu_sc as plsc

N = 20000
G = 5000
NC = 81
ND = 324

NW = 32
SEGB = 160
RCH = 64
NCB = NC // 16
NDB = ND // 16


CTAIL = NC - 16
DTAIL = ND - 16


def _sc_body(S_hbm, J_hbm, C_hbm, D_hbm, bounds_hbm,
             MC_hbm, MD_hbm,
             bounds_v, Sv, Jv, Cv, Dv, AC, AD, ws_v):
    c = lax.axis_index("c")
    s = lax.axis_index("s")
    wid = s * 2 + c
    g0 = pl.multiple_of(jnp.minimum(wid * SEGB, G - SEGB), 8)
    zero16 = jnp.zeros((16,), jnp.float32)
    lane = lax.iota(jnp.int32, 16)
    cmask = lane == 0
    ccol = jnp.full((16,), NC - 1, jnp.int32)
    dmask = lane < (ND - 16 * NDB)
    dcol = jnp.minimum(16 * NDB + lane, ND - 1)

    def zrow(g, carry):
        gv = jnp.full((16,), g, jnp.int32)
        for k in range(NCB):
            AC[g, pl.ds(k * 16, 16)] = zero16
        plsc.store_scatter(AC, [gv, ccol], zero16, mask=cmask)
        for k in range(NDB):
            AD[g, pl.ds(k * 16, 16)] = zero16
        plsc.store_scatter(AD, [gv, dcol], zero16, mask=dmask)
        return carry
    lax.fori_loop(0, SEGB, zrow, 0)
    for t in range(SEGB // 16):
        ws_v[pl.ds(t * 16, 16)] = zero16

    pltpu.sync_copy(bounds_hbm, bounds_v)
    b16 = bounds_v[wid, :]
    lo = b16[0]
    hi = b16[1]
    lo8 = lo - lax.rem(lo, 8)
    nch = (hi - lo8 + (RCH - 1)) // RCH

    def chunk(cix, carry):
        r0 = lo8 + cix * RCH
        b = pl.multiple_of(jnp.minimum(r0, N - RCH), 8)
        o = r0 - b
        pltpu.sync_copy(S_hbm.at[pl.ds(b, RCH)], Sv)
        pltpu.sync_copy(J_hbm.at[pl.ds(b, RCH)], Jv)
        pltpu.sync_copy(C_hbm.at[pl.ds(b, RCH), :], Cv)
        pltpu.sync_copy(D_hbm.at[pl.ds(b, RCH), :], Dv)

        def grp(t, gcarry):
            base = pl.multiple_of(t * 16, 8)
            jv16 = Jv[pl.ds(base, 16)]
            sv16 = Sv[pl.ds(base, 16)]
            rowid = base + lane
            inr = (jv16 >= g0) & (jv16 < g0 + SEGB) & (rowid >= o)
            sv16m = jnp.where(inr, sv16, 0.0)
            sidx16 = jnp.clip(jv16 - g0, 0, SEGB - 1)
            plsc.addupdate_scatter(ws_v, [sidx16], sv16m)
            for l in range(16):
                lv = jnp.full((16,), l, jnp.int32)
                sidxv = sidx16[lv]
                svec = sv16m[lv]
                i = base + l
                iv = jnp.full((16,), i, jnp.int32)
                xs_c = [Cv[i, pl.ds(k * 16, 16)] for k in range(NCB)]
                xs_d = [Dv[i, pl.ds(k * 16, 16)] for k in range(NDB)]
                xc = plsc.load_gather(Cv, [iv, ccol], mask=cmask)
                xd = plsc.load_gather(Dv, [iv, dcol], mask=dmask)
                ws_c = [x * svec for x in xs_c]
                ws_d = [x * svec for x in xs_d]
                for k in range(NCB):
                    plsc.addupdate_scatter(AC, [sidxv, lane + (k * 16)], ws_c[k])
                plsc.addupdate_scatter(AC, [sidxv, ccol], xc * svec, mask=cmask)
                for k in range(NDB):
                    plsc.addupdate_scatter(AD, [sidxv, lane + (k * 16)], ws_d[k])
                plsc.addupdate_scatter(AD, [sidxv, dcol], xd * svec, mask=dmask)
            return gcarry
        lax.fori_loop(0, RCH // 16, grp, 0)
        return carry
    lax.fori_loop(0, nch, chunk, 0)

    def drow(g, carry):
        gv = jnp.full((16,), g, jnp.int32)
        rv = 1.0 / (plsc.load_gather(ws_v, [gv]) + 1e-6)
        for k in range(NCB):
            ksl = pl.ds(k * 16, 16)
            AC[g, ksl] = AC[g, ksl] * rv
        xc = plsc.load_gather(AC, [gv, ccol], mask=cmask)
        plsc.store_scatter(AC, [gv, ccol], xc * rv, mask=cmask)
        for k in range(NDB):
            ksl = pl.ds(k * 16, 16)
            AD[g, ksl] = AD[g, ksl] * rv
        xd = plsc.load_gather(AD, [gv, dcol], mask=dmask)
        plsc.store_scatter(AD, [gv, dcol], xd * rv, mask=dmask)
        return carry
    lax.fori_loop(0, SEGB, drow, 0)

    pltpu.sync_copy(AC, MC_hbm.at[pl.ds(g0, SEGB), :])
    pltpu.sync_copy(AD, MD_hbm.at[pl.ds(g0, SEGB), :])


@jax.jit
def _roimerge_sc(S, J32, C, D, bounds):
    mesh = plsc.VectorSubcoreMesh(core_axis_name="c", subcore_axis_name="s")
    run = functools.partial(
        pl.kernel,
        out_type=(
            jax.ShapeDtypeStruct((G, NC), jnp.float32),
            jax.ShapeDtypeStruct((G, ND), jnp.float32),
        ),
        mesh=mesh,
        scratch_types=[
            pltpu.VMEM((NW, 16), jnp.int32),
            pltpu.VMEM((RCH,), jnp.float32),
            pltpu.VMEM((RCH,), jnp.int32),
            pltpu.VMEM((RCH, NC), jnp.float32),
            pltpu.VMEM((RCH, ND), jnp.float32),
            pltpu.VMEM((SEGB, NC), jnp.float32),
            pltpu.VMEM((SEGB, ND), jnp.float32),
            pltpu.VMEM((SEGB,), jnp.float32),
        ],
        compiler_params=pltpu.CompilerParams(needs_layout_passes=False),
    )(_sc_body)
    return run(S, J32, C, D, bounds)


def kernel(S, J, C, D, P):
    J32 = J.astype(jnp.int32)
    g0s = np.minimum(np.arange(NW, dtype=np.int32) * SEGB, G - SEGB).astype(np.int32)
    qs = np.stack([g0s, g0s + SEGB], axis=1)
    lohi = jnp.sum(J32[None, None, :] < jnp.asarray(qs)[:, :, None],
                   axis=-1, dtype=jnp.int32)
    bounds = jnp.pad(lohi, ((0, 0), (0, 14)))
    MC, MD = _roimerge_sc(S, J32, C, D, bounds)
    return (MC, MD)


# speedup vs baseline: 3.5031x; 1.0002x over previous
import functools

import jax
import jax.numpy as jnp
import numpy as np
from jax import lax
from jax.experimental import pallas as pl
from jax.experimental.pallas import tpu as pltpu
from jax.experimental.pallas import tpu_sc as plsc

N = 20000
G = 5000
NC = 81
ND = 324

NW = 32
SEGB = 160
RCH = 64
NCB = NC // 16
NDB = ND // 16


CTAIL = NC - 16
DTAIL = ND - 16


def _sc_body(S_hbm, J_hbm, C_hbm, D_hbm, bounds_hbm,
             MC_hbm, MD_hbm,
             bounds_v, Sv, Jv, Cv, Dv, AC, AD, ws_v):
    c = lax.axis_index("c")
    s = lax.axis_index("s")
    wid = s * 2 + c
    g0 = pl.multiple_of(jnp.minimum(wid * SEGB, G - SEGB), 8)
    zero16 = jnp.zeros((16,), jnp.float32)
    lane = lax.iota(jnp.int32, 16)
    cmask = lane == 0
    ccol = jnp.full((16,), NC - 1, jnp.int32)
    dmask = lane < (ND - 16 * NDB)
    dcol = jnp.minimum(16 * NDB + lane, ND - 1)

    def zrow(g, carry):
        gv = jnp.full((16,), g, jnp.int32)
        for k in range(NCB):
            AC[g, pl.ds(k * 16, 16)] = zero16
        plsc.store_scatter(AC, [gv, ccol], zero16, mask=cmask)
        for k in range(NDB):
            AD[g, pl.ds(k * 16, 16)] = zero16
        plsc.store_scatter(AD, [gv, dcol], zero16, mask=dmask)
        return carry
    lax.fori_loop(0, SEGB, zrow, 0)
    for t in range(SEGB // 16):
        ws_v[pl.ds(t * 16, 16)] = zero16

    pltpu.sync_copy(bounds_hbm, bounds_v)
    b16 = bounds_v[wid, :]
    lo = b16[0]
    hi = b16[1]
    lo8 = lo - lax.rem(lo, 8)
    nch = (hi - lo8 + (RCH - 1)) // RCH

    def chunk(cix, carry):
        r0 = lo8 + cix * RCH
        b = pl.multiple_of(jnp.minimum(r0, N - RCH), 8)
        o = r0 - b
        pltpu.sync_copy(S_hbm.at[pl.ds(b, RCH)], Sv)
        pltpu.sync_copy(J_hbm.at[pl.ds(b, RCH)], Jv)
        pltpu.sync_copy(C_hbm.at[pl.ds(b, RCH), :], Cv)
        pltpu.sync_copy(D_hbm.at[pl.ds(b, RCH), :], Dv)

        def grp(t, gcarry):
            base = pl.multiple_of(t * 16, 8)
            jv16 = Jv[pl.ds(base, 16)]
            sv16 = Sv[pl.ds(base, 16)]
            rowid = base + lane
            inr = (jv16 >= g0) & (jv16 < g0 + SEGB) & (rowid >= o)
            sv16m = jnp.where(inr, sv16, 0.0)
            sidx16 = jnp.clip(jv16 - g0, 0, SEGB - 1)
            plsc.addupdate_scatter(ws_v, [sidx16], sv16m)
            for l in range(16):
                lv = jnp.full((16,), l, jnp.int32)
                sidxv = sidx16[lv]
                svec = sv16m[lv]
                i = base + l
                iv = jnp.full((16,), i, jnp.int32)
                xs_c = [Cv[i, pl.ds(k * 16, 16)] for k in range(NCB)]
                xs_d = [Dv[i, pl.ds(k * 16, 16)] for k in range(NDB)]
                xc = plsc.load_gather(Cv, [iv, ccol], mask=cmask)
                xd = plsc.load_gather(Dv, [iv, dcol], mask=dmask)
                ws_c = [x * svec for x in xs_c]
                ws_d = [x * svec for x in xs_d]
                for k in range(NCB):
                    plsc.addupdate_scatter(AC, [sidxv, lane + (k * 16)], ws_c[k])
                plsc.addupdate_scatter(AC, [sidxv, ccol], xc * svec, mask=cmask)
                for k in range(NDB):
                    plsc.addupdate_scatter(AD, [sidxv, lane + (k * 16)], ws_d[k])
                plsc.addupdate_scatter(AD, [sidxv, dcol], xd * svec, mask=dmask)
            return gcarry
        lax.fori_loop(0, RCH // 16, grp, 0)
        return carry
    lax.fori_loop(0, nch, chunk, 0)

    def drow(g, carry):
        gv = jnp.full((16,), g, jnp.int32)
        rv = 1.0 / (plsc.load_gather(ws_v, [gv]) + 1e-6)
        for k in range(NCB):
            ksl = pl.ds(k * 16, 16)
            AC[g, ksl] = AC[g, ksl] * rv
        xc = plsc.load_gather(AC, [gv, ccol], mask=cmask)
        plsc.store_scatter(AC, [gv, ccol], xc * rv, mask=cmask)
        for k in range(NDB):
            ksl = pl.ds(k * 16, 16)
            AD[g, ksl] = AD[g, ksl] * rv
        xd = plsc.load_gather(AD, [gv, dcol], mask=dmask)
        plsc.store_scatter(AD, [gv, dcol], xd * rv, mask=dmask)
        return carry
    lax.fori_loop(0, SEGB, drow, 0)

    pltpu.sync_copy(AC, MC_hbm.at[pl.ds(g0, SEGB), :])
    pltpu.sync_copy(AD, MD_hbm.at[pl.ds(g0, SEGB), :])


@jax.jit
def _roimerge_sc(S, J32, C, D, bounds):
    mesh = plsc.VectorSubcoreMesh(core_axis_name="c", subcore_axis_name="s")
    run = functools.partial(
        pl.kernel,
        out_type=(
            jax.ShapeDtypeStruct((G, NC), jnp.float32),
            jax.ShapeDtypeStruct((G, ND), jnp.float32),
        ),
        mesh=mesh,
        scratch_types=[
            pltpu.VMEM((NW, 16), jnp.int32),
            pltpu.VMEM((RCH,), jnp.float32),
            pltpu.VMEM((RCH,), jnp.int32),
            pltpu.VMEM((RCH, NC), jnp.float32),
            pltpu.VMEM((RCH, ND), jnp.float32),
            pltpu.VMEM((SEGB, NC), jnp.float32),
            pltpu.VMEM((SEGB, ND), jnp.float32),
            pltpu.VMEM((SEGB,), jnp.float32),
        ],
        compiler_params=pltpu.CompilerParams(needs_layout_passes=False,
                                             use_tc_tiling_on_sc=True),
    )(_sc_body)
    return run(S, J32, C, D, bounds)


def kernel(S, J, C, D, P):
    J32 = J.astype(jnp.int32)
    g0s = np.minimum(np.arange(NW, dtype=np.int32) * SEGB, G - SEGB).astype(np.int32)
    qs = np.stack([g0s, g0s + SEGB], axis=1)
    lohi = jnp.sum(J32[None, None, :] < jnp.asarray(qs)[:, :, None],
                   axis=-1, dtype=jnp.int32)
    bounds = jnp.pad(lohi, ((0, 0), (0, 14)))
    MC, MD = _roimerge_sc(S, J32, C, D, bounds)
    return (MC, MD)
